# Initial kernel scaffold; baseline (speedup 1.0000x reference)
import jax
import jax.numpy as jnp
from jax.experimental import pallas as pl


def kernel(class_logits, box_regression, proposals):
    raise NotImplementedError("write your pallas kernel here")


# speedup vs baseline: 34.7379x; 34.7379x over previous
import functools
import math

import jax
import jax.numpy as jnp
from jax import lax
from jax.experimental import pallas as pl
from jax.experimental.pallas import tpu as pltpu

N = 5000
NUM_CLASSES = 91
C_PAD = 96
P_SUB = 8
P_LANE = 640
P_PAD = P_SUB * P_LANE
SCORE_THRESH = 0.05
NMS_THRESH = 0.5
DETS_PER_IMG = 100
IMG_H = 800.0
IMG_W = 800.0
BBOX_CLIP = math.log(1000.0 / 16.0)
NEG = -1.0


def _nms_body(logits_ref, dx_ref, dy_ref, dw_ref, dh_ref, prop_ref,
              out_ref, s_ref, x1_ref, y1_ref, x2_ref, y2_ref):
    px1 = prop_ref[0]
    py1 = prop_ref[1]
    px2 = prop_ref[2]
    py2 = prop_ref[3]
    w = px2 - px1
    h = py2 - py1
    cx = px1 + 0.5 * w
    cy = py1 + 0.5 * h

    logits = logits_ref[...]
    lmax = jnp.max(logits[:NUM_CLASSES], axis=0)
    e = jnp.exp(logits - lmax[None])
    denom = jnp.sum(e[:NUM_CLASSES], axis=0)
    scores = e / denom[None]

    dx = dx_ref[...] * (1.0 / 10.0)
    dy = dy_ref[...] * (1.0 / 10.0)
    dw = jnp.minimum(dw_ref[...] * (1.0 / 5.0), BBOX_CLIP)
    dh = jnp.minimum(dh_ref[...] * (1.0 / 5.0), BBOX_CLIP)
    pcx = dx * w[None] + cx[None]
    pcy = dy * h[None] + cy[None]
    pw = jnp.exp(dw) * w[None]
    ph = jnp.exp(dh) * h[None]
    x1 = jnp.clip(pcx - 0.5 * pw, 0.0, IMG_W)
    y1 = jnp.clip(pcy - 0.5 * ph, 0.0, IMG_H)
    x2 = jnp.clip(pcx + 0.5 * pw, 0.0, IMG_W)
    y2 = jnp.clip(pcy + 0.5 * ph, 0.0, IMG_H)

    pidx = (lax.broadcasted_iota(jnp.int32, (P_SUB, P_LANE), 0) * P_LANE
            + lax.broadcasted_iota(jnp.int32, (P_SUB, P_LANE), 1))
    crow = lax.broadcasted_iota(jnp.int32, (C_PAD, P_SUB, P_LANE), 0)
    valid = ((scores > SCORE_THRESH)
             & ((x2 - x1) >= 0.01)
             & ((y2 - y1) >= 0.01)
             & (pidx[None] < N)
             & (crow >= 1) & (crow < NUM_CLASSES))
    s = jnp.where(valid, scores, NEG)

    s_ref[...] = s
    x1_ref[...] = x1
    y1_ref[...] = y1
    x2_ref[...] = x2
    y2_ref[...] = y2

    rowmax = jnp.max(jnp.max(s, axis=2), axis=1)
    m2 = jnp.concatenate(
        [rowmax, jnp.full((128 - C_PAD,), NEG, jnp.float32)], axis=0
    ).reshape(1, 128)

    li = lax.broadcasted_iota(jnp.int32, (1, 128), 1)
    arow = lax.broadcasted_iota(jnp.int32, (128, 128), 0)
    BIG = jnp.int32(2 ** 30)

    def body(t, carry):
        m2, acc = carry
        vstar = jnp.max(m2)
        cstar = jnp.min(jnp.where(m2 == vstar, li, BIG))
        cond = vstar > 0.0
        c = jnp.where(cond, cstar, 0)

        srow = s_ref[c]
        x1r = x1_ref[c]
        y1r = y1_ref[c]
        x2r = x2_ref[c]
        y2r = y2_ref[c]

        irow = jnp.min(jnp.where(srow == vstar, pidx, BIG))
        selmask = pidx == irow
        bx1 = jnp.sum(jnp.where(selmask, x1r, 0.0))
        by1 = jnp.sum(jnp.where(selmask, y1r, 0.0))
        bx2 = jnp.sum(jnp.where(selmask, x2r, 0.0))
        by2 = jnp.sum(jnp.where(selmask, y2r, 0.0))

        xx1 = jnp.maximum(bx1, x1r)
        yy1 = jnp.maximum(by1, y1r)
        xx2 = jnp.minimum(bx2, x2r)
        yy2 = jnp.minimum(by2, y2r)
        inter = (jnp.clip(xx2 - xx1, 0.0) * jnp.clip(yy2 - yy1, 0.0))
        barea = (bx2 - bx1) * (by2 - by1)
        areas = (x2r - x1r) * (y2r - y1r)
        iou = inter / (barea + areas - inter)
        supp = (iou > NMS_THRESH) | selmask
        new_srow = jnp.where(supp, NEG, srow)
        s_ref[c] = jnp.where(cond, new_srow, srow)

        newmax = jnp.max(new_srow)
        m2 = jnp.where(cond & (li == c), newmax, m2)

        packed = (jnp.where(li == 0, bx1, 0.0)
                  + jnp.where(li == 1, by1, 0.0)
                  + jnp.where(li == 2, bx2, 0.0)
                  + jnp.where(li == 3, by2, 0.0)
                  + jnp.where(li == 4, vstar, 0.0)
                  + jnp.where(li == 5, c.astype(jnp.float32), 0.0))
        acc = jnp.where((arow == t) & cond, packed, acc)
        return m2, acc

    acc0 = jnp.zeros((128, 128), jnp.float32)
    _, acc = lax.fori_loop(0, DETS_PER_IMG, body, (m2, acc0))
    out_ref[...] = acc


@functools.partial(jax.jit)
def kernel(class_logits, box_regression, proposals):
    f32 = jnp.float32
    lt = jnp.zeros((C_PAD, P_PAD), f32)
    lt = lt.at[:NUM_CLASSES, :N].set(class_logits.T)
    lt = lt.reshape(C_PAD, P_SUB, P_LANE)

    br = box_regression.reshape(N, NUM_CLASSES, 4)
    planes = []
    for k in range(4):
        pk = jnp.zeros((C_PAD, P_PAD), f32)
        pk = pk.at[:NUM_CLASSES, :N].set(br[:, :, k].T)
        planes.append(pk.reshape(C_PAD, P_SUB, P_LANE))
    dx_t, dy_t, dw_t, dh_t = planes

    prop = jnp.zeros((4, P_PAD), f32)
    prop = prop.at[:, :N].set(proposals.T)
    prop = prop.reshape(4, P_SUB, P_LANE)

    scratch = pltpu.VMEM((C_PAD, P_SUB, P_LANE), f32)
    packed = pl.pallas_call(
        _nms_body,
        out_shape=jax.ShapeDtypeStruct((128, 128), f32),
        scratch_shapes=[scratch] * 5,
    )(lt, dx_t, dy_t, dw_t, dh_t, prop)

    out_boxes = packed[:DETS_PER_IMG, 0:4]
    out_scores = packed[:DETS_PER_IMG, 4]
    out_labels = packed[:DETS_PER_IMG, 5].astype(jnp.int32)
    return out_boxes, out_scores, out_labels
